# Initial kernel scaffold; baseline (speedup 1.0000x reference)
import jax
import jax.numpy as jnp
from jax.experimental import pallas as pl


def kernel(edges, W, b, edge_ids, combine_method):
    raise NotImplementedError("write your pallas kernel here")


# speedup vs baseline: 4.7090x; 4.7090x over previous
import functools

import jax
import jax.numpy as jnp
from jax import lax
from jax.experimental import pallas as pl
from jax.experimental.pallas import tpu as pltpu
from jax.experimental.pallas import tpu_sc as plsc


def _proj_body(x_ref, w_ref, b_ref, o_ref):
    o_ref[...] = (
        jnp.dot(x_ref[...], w_ref[...], preferred_element_type=jnp.float32)
        + b_ref[...]
    )


def _project(edges, w1t, b):
    E, D = edges.shape
    OUT = w1t.shape[1]
    BLK = 4000
    assert E % BLK == 0
    return pl.pallas_call(
        _proj_body,
        grid=(E // BLK,),
        in_specs=[
            pl.BlockSpec((BLK, D), lambda i: (i, 0)),
            pl.BlockSpec((D, OUT), lambda i: (0, 0)),
            pl.BlockSpec((1, OUT), lambda i: (0, 0)),
        ],
        out_specs=pl.BlockSpec((BLK, OUT), lambda i: (i, 0)),
        out_shape=jax.ShapeDtypeStruct((E, OUT), jnp.float32),
    )(edges, w1t, b.reshape(1, OUT))


def _gather_rows(table, idx, chunk):
    B = idx.shape[0]
    OUT = table.shape[1]
    info = plsc.get_sparse_core_info()
    nw = info.num_cores * info.num_subcores
    rows_per_w = B // nw
    assert B % nw == 0 and rows_per_w % chunk == 0
    n_chunks = rows_per_w // chunk
    mesh = plsc.VectorSubcoreMesh(core_axis_name="c", subcore_axis_name="s")

    @functools.partial(
        pl.kernel,
        mesh=mesh,
        compiler_params=pltpu.CompilerParams(use_tc_tiling_on_sc=False),
        out_type=jax.ShapeDtypeStruct((B, OUT), jnp.float32),
        scratch_types=[
            pltpu.VMEM((chunk,), jnp.int32),
            pltpu.VMEM((chunk, OUT), jnp.float32),
            pltpu.SemaphoreType.DMA,
        ],
    )
    def k(table_hbm, idx_hbm, out_hbm, idx_v, rows_v, sem):
        wid = lax.axis_index("s") * info.num_cores + lax.axis_index("c")
        base = wid * rows_per_w

        def body(c, _):
            off = base + c * chunk
            pltpu.sync_copy(idx_hbm.at[pl.ds(off, chunk)], idx_v)
            pltpu.async_copy(table_hbm.at[idx_v], rows_v, sem).wait()
            pltpu.sync_copy(rows_v, out_hbm.at[pl.ds(off, chunk)])
            return 0

        lax.fori_loop(0, n_chunks, body, 0)

    return k(table, idx)


def kernel(edges, W, b, edge_ids, combine_method):
    E, D = edges.shape
    OUT = W.shape[0]
    w1t = W[:, :D].T
    proj = _project(edges, w1t, b)
    ids2 = jnp.stack([edge_ids, edge_ids[::-1]], axis=1).reshape(-1)
    gathered = _gather_rows(proj, ids2, chunk=2000)
    return gathered.reshape(E, 2 * OUT)


# speedup vs baseline: 6.8234x; 1.4490x over previous
import functools

import jax
import jax.numpy as jnp
from jax import lax
from jax.experimental import pallas as pl
from jax.experimental.pallas import tpu as pltpu
from jax.experimental.pallas import tpu_sc as plsc


def _proj_body(x_ref, w_ref, b_ref, o_ref):
    o_ref[...] = (
        jnp.dot(x_ref[...], w_ref[...], preferred_element_type=jnp.float32)
        + b_ref[...]
    )


def _project(edges, w1t, b):
    E, D = edges.shape
    OUT = w1t.shape[1]
    BLK = 4000
    assert E % BLK == 0
    return pl.pallas_call(
        _proj_body,
        grid=(E // BLK,),
        in_specs=[
            pl.BlockSpec((BLK, D), lambda i: (i, 0)),
            pl.BlockSpec((D, OUT), lambda i: (0, 0)),
            pl.BlockSpec((1, OUT), lambda i: (0, 0)),
        ],
        out_specs=pl.BlockSpec((BLK, OUT), lambda i: (i, 0)),
        out_shape=jax.ShapeDtypeStruct((E, OUT), jnp.float32),
    )(edges, w1t, b.reshape(1, OUT))


def _gather_both(table, ids, chunk):
    E, OUT = table.shape
    info = plsc.get_sparse_core_info()
    nw = info.num_cores * info.num_subcores
    L = info.num_lanes
    P = E // nw
    assert E % nw == 0 and P % L == 0 and (2 * P) % chunk == 0
    n_chunks = (2 * P) // chunk
    mesh = plsc.VectorSubcoreMesh(core_axis_name="c", subcore_axis_name="s")

    @functools.partial(
        pl.kernel,
        mesh=mesh,
        compiler_params=pltpu.CompilerParams(
            use_tc_tiling_on_sc=False, needs_layout_passes=False
        ),
        out_type=jax.ShapeDtypeStruct((2 * E, OUT), jnp.float32),
        scratch_types=[
            pltpu.VMEM((P,), jnp.int32),
            pltpu.VMEM((P,), jnp.int32),
            pltpu.VMEM((2 * P,), jnp.int32),
            pltpu.VMEM((chunk, OUT), jnp.float32),
            pltpu.SemaphoreType.DMA,
        ],
    )
    def k(table_hbm, ids_hbm, out_hbm, idxf_v, idxr_v, inter_v, rows_v, sem):
        wid = lax.axis_index("s") * info.num_cores + lax.axis_index("c")
        p0 = wid * P
        pltpu.sync_copy(ids_hbm.at[pl.ds(p0, P)], idxf_v)
        pltpu.sync_copy(ids_hbm.at[pl.ds(E - p0 - P, P)], idxr_v)

        iota = lax.iota(jnp.int32, L)

        def build(kk, _):
            m = kk * L + iota
            vf = idxf_v[pl.ds(kk * L, L)]
            plsc.store_scatter(inter_v, [m * 2], vf)
            vr = idxr_v[pl.ds(kk * L, L)]
            plsc.store_scatter(inter_v, [(P - 1 - m) * 2 + 1], vr)
            return 0

        lax.fori_loop(0, P // L, build, 0)

        def body(c, _):
            off = c * chunk
            pltpu.async_copy(
                table_hbm.at[inter_v.at[pl.ds(off, chunk)]], rows_v, sem
            ).wait()
            pltpu.sync_copy(rows_v, out_hbm.at[pl.ds(2 * p0 + off, chunk)])
            return 0

        lax.fori_loop(0, n_chunks, body, 0)

    return k(table, ids)


def kernel(edges, W, b, edge_ids, combine_method):
    E, D = edges.shape
    OUT = W.shape[0]
    w1t = W[:, :D].T
    proj = _project(edges, w1t, b)
    gathered = _gather_both(proj, edge_ids, chunk=2000)
    return gathered.reshape(E, 2 * OUT)
